# baseline (speedup 1.0000x reference)
import functools

import jax
import jax.numpy as jnp
from jax import lax
from jax.experimental import pallas as pl
from jax.experimental.pallas import tpu as pltpu
from jax.experimental.pallas import tpu_sc as plsc

_N = 10000
_E = 320000
_H = 128
_BN_EPS = 1e-5

_NC = 2
_NS = 16
_NW = _NC * _NS
_EPW = _E // _NW
_K = 80
_CH = 5
_CE = _EPW // _CH
_NP = 10240
_RPT = _NP // _NS

_ROWS = 2000
_NBLK = _N // _ROWS


def _sc_aggregate(x, src1d, dst1d, zrows):
    mesh = plsc.VectorSubcoreMesh(core_axis_name="c", subcore_axis_name="s")

    @functools.partial(
        pl.kernel,
        out_type=jax.ShapeDtypeStruct((_NC, _NP, _H), jnp.float32),
        mesh=mesh,
        scratch_types=[
            pltpu.VMEM((_CE,), jnp.int32),
            pltpu.VMEM((_CE,), jnp.int32),
            pltpu.VMEM((_CE,), jnp.int32),
            pltpu.VMEM((_CE,), jnp.int32),
            pltpu.VMEM((_K, _H), jnp.float32),
            pltpu.VMEM((_K, _H), jnp.float32),
            pltpu.VMEM_SHARED((_NP, _H), jnp.float32),
            pltpu.SemaphoreType.DMA,
            pltpu.SemaphoreType.DMA,
            pltpu.SemaphoreType.DMA,
        ],
    )
    def agg_kernel(x_hbm, s_hbm, d_hbm, z_hbm, out_hbm,
                   srcv0, srcv1, dstv0, dstv1, bufa, bufb, acc,
                   sema, semb, semi):
        c = lax.axis_index("c")
        s = lax.axis_index("s")
        wid = s * _NC + c
        sbase = wid * _EPW
        dbase = wid * _EPW

        pltpu.sync_copy(s_hbm.at[pl.ds(sbase, _CE)], srcv0)
        pltpu.sync_copy(d_hbm.at[pl.ds(dbase, _CE)], dstv0)
        pltpu.async_copy(s_hbm.at[pl.ds(sbase + _CE, _CE)], srcv1, semi)
        pltpu.async_copy(d_hbm.at[pl.ds(dbase + _CE, _CE)], dstv1, semi)
        pltpu.async_copy(x_hbm.at[srcv0.at[pl.ds(0, _K)]], bufa, sema)
        pltpu.async_copy(x_hbm.at[srcv0.at[pl.ds(_K, _K)]], bufb, semb)
        pltpu.sync_copy(z_hbm, acc.at[pl.ds(s * _RPT, _RPT)])
        plsc.subcore_barrier()

        npair = _CE // _K // 2

        for ch in range(_CH):
            sv, dv = (srcv0, dstv0) if ch % 2 == 0 else (srcv1, dstv1)
            nsv, ndv = (srcv1, dstv1) if ch % 2 == 0 else (srcv0, dstv0)
            A, B = (bufa, bufb) if ch % 2 == 0 else (bufb, bufa)
            sA, sB = (sema, semb) if ch % 2 == 0 else (semb, sema)

            def pair(i, carry2, sv=sv, dv=dv, A=A, B=B, sA=sA, sB=sB):
                o = 2 * i * _K
                pltpu.make_async_copy(x_hbm.at[sv.at[pl.ds(o, _K)]], A, sA).wait()
                pltpu.sync_copy(A, acc.at[dv.at[pl.ds(o, _K)]], add=True)
                pltpu.async_copy(x_hbm.at[sv.at[pl.ds(o + 2 * _K, _K)]], A, sA)
                pltpu.make_async_copy(
                    x_hbm.at[sv.at[pl.ds(o + _K, _K)]], B, sB).wait()
                pltpu.sync_copy(B, acc.at[dv.at[pl.ds(o + _K, _K)]], add=True)
                pltpu.async_copy(x_hbm.at[sv.at[pl.ds(o + 3 * _K, _K)]], B, sB)
                return carry2

            lax.fori_loop(0, npair - 1, pair, 0)
            o22, o23, o24 = 22 * _K, 23 * _K, 24 * _K
            if ch + 1 < _CH:
                pltpu.make_async_copy(
                    s_hbm.at[pl.ds(sbase + (ch + 1) * _CE, _CE)], nsv, semi).wait()
                pltpu.make_async_copy(
                    d_hbm.at[pl.ds(dbase + (ch + 1) * _CE, _CE)], ndv, semi).wait()
                pltpu.make_async_copy(x_hbm.at[sv.at[pl.ds(o22, _K)]], A, sA).wait()
                pltpu.sync_copy(A, acc.at[dv.at[pl.ds(o22, _K)]], add=True)
                pltpu.async_copy(x_hbm.at[sv.at[pl.ds(o24, _K)]], A, sA)
                pltpu.make_async_copy(x_hbm.at[sv.at[pl.ds(o23, _K)]], B, sB).wait()
                pltpu.sync_copy(B, acc.at[dv.at[pl.ds(o23, _K)]], add=True)
                pltpu.async_copy(x_hbm.at[nsv.at[pl.ds(0, _K)]], B, sB)
                pltpu.make_async_copy(x_hbm.at[sv.at[pl.ds(o24, _K)]], A, sA).wait()
                pltpu.sync_copy(A, acc.at[dv.at[pl.ds(o24, _K)]], add=True)
                pltpu.async_copy(x_hbm.at[nsv.at[pl.ds(_K, _K)]], A, sA)
                if ch + 2 < _CH:
                    pltpu.async_copy(
                        s_hbm.at[pl.ds(sbase + (ch + 2) * _CE, _CE)], sv, semi)
                    pltpu.async_copy(
                        d_hbm.at[pl.ds(dbase + (ch + 2) * _CE, _CE)], dv, semi)
            else:
                pltpu.make_async_copy(x_hbm.at[sv.at[pl.ds(o22, _K)]], A, sA).wait()
                pltpu.sync_copy(A, acc.at[dv.at[pl.ds(o22, _K)]], add=True)
                pltpu.async_copy(x_hbm.at[sv.at[pl.ds(o24, _K)]], A, sA)
                pltpu.make_async_copy(x_hbm.at[sv.at[pl.ds(o23, _K)]], B, sB).wait()
                pltpu.sync_copy(B, acc.at[dv.at[pl.ds(o23, _K)]], add=True)
                pltpu.make_async_copy(x_hbm.at[sv.at[pl.ds(o24, _K)]], A, sA).wait()
                pltpu.sync_copy(A, acc.at[dv.at[pl.ds(o24, _K)]], add=True)

        plsc.subcore_barrier()
        pltpu.sync_copy(acc.at[pl.ds(s * _RPT, _RPT)],
                        out_hbm.at[c, pl.ds(s * _RPT, _RPT)])

    return agg_kernel(x, src1d, dst1d, zrows)


def _bn_affine(sum_r, sq_r, g_r, bt_r):
    mean = sum_r[...] * (1.0 / _N)
    var = sq_r[...] * (1.0 / _N) - mean * mean
    inv = lax.rsqrt(var + _BN_EPS)
    sc = g_r[...] * inv
    sh = bt_r[...] - mean * sc
    return sc, sh


def _mlp_body(scale_r, x_r, a0_r, a1_r, w1_r, b1_r, g1_r, bt1_r,
              w2_r, b2_r, g3_r, bt3_r, o_r,
              z_scr, z2_scr, sum1, sq1, sum2, sq2):
    p = pl.program_id(0)
    i = pl.program_id(1)
    rows = pl.ds(i * _ROWS, _ROWS)

    @pl.when(p == 0)
    def _():
        hin = x_r[...] * scale_r[0, 0] + a0_r[0] + a1_r[0]
        z = jnp.dot(hin, w1_r[...], preferred_element_type=jnp.float32) + b1_r[...]
        z_scr[rows, :] = z.astype(jnp.bfloat16)
        ps = jnp.sum(z, axis=0, keepdims=True)
        pq = jnp.sum(z * z, axis=0, keepdims=True)

        @pl.when(i == 0)
        def _():
            sum1[...] = ps
            sq1[...] = pq

        @pl.when(i > 0)
        def _():
            sum1[...] += ps
            sq1[...] += pq

    @pl.when(p == 1)
    def _():
        sc, sh = _bn_affine(sum1, sq1, g1_r, bt1_r)
        a = jnp.maximum(z_scr[rows, :].astype(jnp.float32) * sc + sh, 0.0)
        z2 = jnp.dot(a, w2_r[...], preferred_element_type=jnp.float32) + b2_r[...]
        z2_scr[rows, :] = z2.astype(jnp.bfloat16)
        ps = jnp.sum(z2, axis=0, keepdims=True)
        pq = jnp.sum(z2 * z2, axis=0, keepdims=True)

        @pl.when(i == 0)
        def _():
            sum2[...] = ps
            sq2[...] = pq

        @pl.when(i > 0)
        def _():
            sum2[...] += ps
            sq2[...] += pq

    @pl.when(p == 2)
    def _():
        sc, sh = _bn_affine(sum2, sq2, g3_r, bt3_r)
        o_r[...] = jnp.maximum(z2_scr[rows, :].astype(jnp.float32) * sc + sh, 0.0)


def _p0_row_spec(cols):
    return pl.BlockSpec((_ROWS, cols),
                        lambda p, i: (jnp.where(p == 0, i, 0), 0))


def _full_spec(rows, cols):
    return pl.BlockSpec((rows, cols), lambda p, i: (0, 0))


def kernel(x, edge_index, eps, W1, b1, g1, bt1, W2, b2, g3, bt3):
    src1d = edge_index[0].astype(jnp.int32)
    dst1d = edge_index[1].astype(jnp.int32)
    zrows = jnp.zeros((_RPT, _H), dtype=jnp.float32)

    parts = _sc_aggregate(x, src1d, dst1d, zrows)

    scale = (1.0 + eps).reshape(1, 1)
    H2 = 2 * _H

    out = pl.pallas_call(
        _mlp_body,
        grid=(3, _NBLK),
        in_specs=[
            pl.BlockSpec(memory_space=pltpu.SMEM),
            _p0_row_spec(_H),
            pl.BlockSpec((1, _ROWS, _H),
                         lambda p, i: (0, jnp.where(p == 0, i, 0), 0)),
            pl.BlockSpec((1, _ROWS, _H),
                         lambda p, i: (1, jnp.where(p == 0, i, 0), 0)),
            _full_spec(_H, H2), _full_spec(1, H2),
            _full_spec(1, H2), _full_spec(1, H2),
            _full_spec(H2, _H), _full_spec(1, _H),
            _full_spec(1, _H), _full_spec(1, _H),
        ],
        out_specs=pl.BlockSpec((_ROWS, _H),
                               lambda p, i: (jnp.where(p == 2, i, 0), 0)),
        out_shape=jax.ShapeDtypeStruct((_N, _H), jnp.float32),
        scratch_shapes=[
            pltpu.VMEM((_N, H2), jnp.bfloat16),
            pltpu.VMEM((_N, _H), jnp.bfloat16),
            pltpu.VMEM((1, H2), jnp.float32),
            pltpu.VMEM((1, H2), jnp.float32),
            pltpu.VMEM((1, _H), jnp.float32),
            pltpu.VMEM((1, _H), jnp.float32),
        ],
    )(scale, x, parts, parts, W1, b1.reshape(1, H2),
      g1.reshape(1, H2), bt1.reshape(1, H2), W2, b2.reshape(1, _H),
      g3.reshape(1, _H), bt3.reshape(1, _H))

    return out


# speedup vs baseline: 1.0712x; 1.0712x over previous
import functools

import jax
import jax.numpy as jnp
from jax import lax
from jax.experimental import pallas as pl
from jax.experimental.pallas import tpu as pltpu
from jax.experimental.pallas import tpu_sc as plsc

_N = 10000
_E = 320000
_H = 128
_BN_EPS = 1e-5

_NC = 2
_NS = 16
_NW = _NC * _NS
_EPW = _E // _NW
_K = 80
_CH = 5
_CE = _EPW // _CH
_NP = 10240
_RPT = _NP // _NS

_ROWS = 2000
_NBLK = _N // _ROWS


def _sc_aggregate(x, e_flat, zrows):
    mesh = plsc.VectorSubcoreMesh(core_axis_name="c", subcore_axis_name="s")

    @functools.partial(
        pl.kernel,
        out_type=jax.ShapeDtypeStruct((_NC, _NP, _H), jnp.float32),
        mesh=mesh,
        scratch_types=[
            pltpu.VMEM((_CE,), jnp.int32),
            pltpu.VMEM((_CE,), jnp.int32),
            pltpu.VMEM((_CE,), jnp.int32),
            pltpu.VMEM((_CE,), jnp.int32),
            pltpu.VMEM((_K, _H), jnp.float32),
            pltpu.VMEM((_K, _H), jnp.float32),
            pltpu.VMEM_SHARED((_NP, _H), jnp.float32),
            pltpu.SemaphoreType.DMA,
            pltpu.SemaphoreType.DMA,
            pltpu.SemaphoreType.DMA,
        ],
    )
    def agg_kernel(x_hbm, e_hbm, z_hbm, out_hbm,
                   srcv0, srcv1, dstv0, dstv1, bufa, bufb, acc,
                   sema, semb, semi):
        c = lax.axis_index("c")
        s = lax.axis_index("s")
        wid = s * _NC + c
        sbase = wid * _EPW
        dbase = _E + wid * _EPW

        pltpu.sync_copy(e_hbm.at[pl.ds(sbase, _CE)], srcv0)
        pltpu.sync_copy(e_hbm.at[pl.ds(dbase, _CE)], dstv0)
        pltpu.async_copy(e_hbm.at[pl.ds(sbase + _CE, _CE)], srcv1, semi)
        pltpu.async_copy(e_hbm.at[pl.ds(dbase + _CE, _CE)], dstv1, semi)
        pltpu.async_copy(x_hbm.at[srcv0.at[pl.ds(0, _K)]], bufa, sema)
        pltpu.async_copy(x_hbm.at[srcv0.at[pl.ds(_K, _K)]], bufb, semb)
        pltpu.sync_copy(z_hbm, acc.at[pl.ds(s * _RPT, _RPT)])
        plsc.subcore_barrier()

        npair = _CE // _K // 2

        for ch in range(_CH):
            sv, dv = (srcv0, dstv0) if ch % 2 == 0 else (srcv1, dstv1)
            nsv, ndv = (srcv1, dstv1) if ch % 2 == 0 else (srcv0, dstv0)
            A, B = (bufa, bufb) if ch % 2 == 0 else (bufb, bufa)
            sA, sB = (sema, semb) if ch % 2 == 0 else (semb, sema)

            def pair(i, carry2, sv=sv, dv=dv, A=A, B=B, sA=sA, sB=sB):
                o = 2 * i * _K
                pltpu.make_async_copy(x_hbm.at[sv.at[pl.ds(o, _K)]], A, sA).wait()
                pltpu.sync_copy(A, acc.at[dv.at[pl.ds(o, _K)]], add=True)
                pltpu.async_copy(x_hbm.at[sv.at[pl.ds(o + 2 * _K, _K)]], A, sA)
                pltpu.make_async_copy(
                    x_hbm.at[sv.at[pl.ds(o + _K, _K)]], B, sB).wait()
                pltpu.sync_copy(B, acc.at[dv.at[pl.ds(o + _K, _K)]], add=True)
                pltpu.async_copy(x_hbm.at[sv.at[pl.ds(o + 3 * _K, _K)]], B, sB)
                return carry2

            lax.fori_loop(0, npair - 1, pair, 0)
            o22, o23, o24 = 22 * _K, 23 * _K, 24 * _K
            if ch + 1 < _CH:
                pltpu.make_async_copy(
                    e_hbm.at[pl.ds(sbase + (ch + 1) * _CE, _CE)], nsv, semi).wait()
                pltpu.make_async_copy(
                    e_hbm.at[pl.ds(dbase + (ch + 1) * _CE, _CE)], ndv, semi).wait()
                pltpu.make_async_copy(x_hbm.at[sv.at[pl.ds(o22, _K)]], A, sA).wait()
                pltpu.sync_copy(A, acc.at[dv.at[pl.ds(o22, _K)]], add=True)
                pltpu.async_copy(x_hbm.at[sv.at[pl.ds(o24, _K)]], A, sA)
                pltpu.make_async_copy(x_hbm.at[sv.at[pl.ds(o23, _K)]], B, sB).wait()
                pltpu.sync_copy(B, acc.at[dv.at[pl.ds(o23, _K)]], add=True)
                pltpu.async_copy(x_hbm.at[nsv.at[pl.ds(0, _K)]], B, sB)
                pltpu.make_async_copy(x_hbm.at[sv.at[pl.ds(o24, _K)]], A, sA).wait()
                pltpu.sync_copy(A, acc.at[dv.at[pl.ds(o24, _K)]], add=True)
                pltpu.async_copy(x_hbm.at[nsv.at[pl.ds(_K, _K)]], A, sA)
                if ch + 2 < _CH:
                    pltpu.async_copy(
                        e_hbm.at[pl.ds(sbase + (ch + 2) * _CE, _CE)], sv, semi)
                    pltpu.async_copy(
                        e_hbm.at[pl.ds(dbase + (ch + 2) * _CE, _CE)], dv, semi)
            else:
                pltpu.make_async_copy(x_hbm.at[sv.at[pl.ds(o22, _K)]], A, sA).wait()
                pltpu.sync_copy(A, acc.at[dv.at[pl.ds(o22, _K)]], add=True)
                pltpu.async_copy(x_hbm.at[sv.at[pl.ds(o24, _K)]], A, sA)
                pltpu.make_async_copy(x_hbm.at[sv.at[pl.ds(o23, _K)]], B, sB).wait()
                pltpu.sync_copy(B, acc.at[dv.at[pl.ds(o23, _K)]], add=True)
                pltpu.make_async_copy(x_hbm.at[sv.at[pl.ds(o24, _K)]], A, sA).wait()
                pltpu.sync_copy(A, acc.at[dv.at[pl.ds(o24, _K)]], add=True)

        plsc.subcore_barrier()
        pltpu.sync_copy(acc.at[pl.ds(s * _RPT, _RPT)],
                        out_hbm.at[c, pl.ds(s * _RPT, _RPT)])

    return agg_kernel(x, e_flat, zrows)


def _bn_affine(sum_r, sq_r, g_r, bt_r):
    mean = sum_r[...] * (1.0 / _N)
    var = sq_r[...] * (1.0 / _N) - mean * mean
    inv = lax.rsqrt(var + _BN_EPS)
    sc = g_r[...] * inv
    sh = bt_r[...] - mean * sc
    return sc, sh


def _mlp_body(scale_r, x_r, a0_r, a1_r, w1_r, b1_r, g1_r, bt1_r,
              w2_r, b2_r, g3_r, bt3_r, o_r,
              z_scr, z2_scr, sum1, sq1, sum2, sq2):
    p = pl.program_id(0)
    i = pl.program_id(1)
    rows = pl.ds(i * _ROWS, _ROWS)

    @pl.when(p == 0)
    def _():
        hin = x_r[...] * scale_r[0, 0] + a0_r[0] + a1_r[0]
        z = jnp.dot(hin, w1_r[...], preferred_element_type=jnp.float32) + b1_r[...]
        z_scr[rows, :] = z.astype(jnp.bfloat16)
        ps = jnp.sum(z, axis=0, keepdims=True)
        pq = jnp.sum(z * z, axis=0, keepdims=True)

        @pl.when(i == 0)
        def _():
            sum1[...] = ps
            sq1[...] = pq

        @pl.when(i > 0)
        def _():
            sum1[...] += ps
            sq1[...] += pq

    @pl.when(p == 1)
    def _():
        sc, sh = _bn_affine(sum1, sq1, g1_r, bt1_r)
        a = jnp.maximum(z_scr[rows, :].astype(jnp.float32) * sc + sh, 0.0)
        z2 = jnp.dot(a, w2_r[...], preferred_element_type=jnp.float32) + b2_r[...]
        z2_scr[rows, :] = z2.astype(jnp.bfloat16)
        ps = jnp.sum(z2, axis=0, keepdims=True)
        pq = jnp.sum(z2 * z2, axis=0, keepdims=True)

        @pl.when(i == 0)
        def _():
            sum2[...] = ps
            sq2[...] = pq

        @pl.when(i > 0)
        def _():
            sum2[...] += ps
            sq2[...] += pq

    @pl.when(p == 2)
    def _():
        sc, sh = _bn_affine(sum2, sq2, g3_r, bt3_r)
        o_r[...] = jnp.maximum(z2_scr[rows, :].astype(jnp.float32) * sc + sh, 0.0)


def _p0_row_spec(cols):
    return pl.BlockSpec((_ROWS, cols),
                        lambda p, i: (jnp.where(p == 0, i, 0), 0))


def _full_spec(rows, cols):
    return pl.BlockSpec((rows, cols), lambda p, i: (0, 0))


def kernel(x, edge_index, eps, W1, b1, g1, bt1, W2, b2, g3, bt3):
    e_flat = edge_index.astype(jnp.int32).reshape(2 * _E)
    zrows = jnp.zeros((_RPT, _H), dtype=jnp.float32)

    parts = _sc_aggregate(x, e_flat, zrows)

    scale = (1.0 + eps).reshape(1, 1)
    H2 = 2 * _H

    out = pl.pallas_call(
        _mlp_body,
        grid=(3, _NBLK),
        in_specs=[
            pl.BlockSpec(memory_space=pltpu.SMEM),
            _p0_row_spec(_H),
            pl.BlockSpec((1, _ROWS, _H),
                         lambda p, i: (0, jnp.where(p == 0, i, 0), 0)),
            pl.BlockSpec((1, _ROWS, _H),
                         lambda p, i: (1, jnp.where(p == 0, i, 0), 0)),
            _full_spec(_H, H2), _full_spec(1, H2),
            _full_spec(1, H2), _full_spec(1, H2),
            _full_spec(H2, _H), _full_spec(1, _H),
            _full_spec(1, _H), _full_spec(1, _H),
        ],
        out_specs=pl.BlockSpec((_ROWS, _H),
                               lambda p, i: (jnp.where(p == 2, i, 0), 0)),
        out_shape=jax.ShapeDtypeStruct((_N, _H), jnp.float32),
        scratch_shapes=[
            pltpu.VMEM((_N, H2), jnp.bfloat16),
            pltpu.VMEM((_N, _H), jnp.bfloat16),
            pltpu.VMEM((1, H2), jnp.float32),
            pltpu.VMEM((1, H2), jnp.float32),
            pltpu.VMEM((1, _H), jnp.float32),
            pltpu.VMEM((1, _H), jnp.float32),
        ],
    )(scale, x, parts, parts, W1, b1.reshape(1, H2),
      g1.reshape(1, H2), bt1.reshape(1, H2), W2, b2.reshape(1, _H),
      g3.reshape(1, _H), bt3.reshape(1, _H))

    return out
